# baseline (speedup 1.0000x reference)
import jax
import jax.numpy as jnp
from jax.experimental import pallas as pl
from jax.experimental.pallas import tpu as pltpu

_NCH = 4
_NPAR = 2


def _build_tap_kernel(n_groups, c_in, length, d_model):
    chunk = length // _NCH

    def body(xt_ref, w_ref, hm_ref, tm_ref, o_hbm, tap_ref, obuf, sems):
        b = pl.program_id(0)
        nb = pl.num_programs(0)
        par = jax.lax.rem(b, _NPAR)

        xt = xt_ref[0]
        sub = jax.lax.broadcasted_iota(jnp.int32, xt.shape, 0)
        for g in range(n_groups):
            d = g - (n_groups - 2)
            rolled = pltpu.roll(xt, (-d) % length, axis=1)
            if g == 0:
                rolled = jnp.where(sub == 7, 1.0, rolled)
            tap_ref[g * 8:(g + 1) * 8, :] = rolled
        tap_ref[:, :128] = tap_ref[:, :128] * hm_ref[...]
        tap_ref[:, length - 128:] = tap_ref[:, length - 128:] * tm_ref[...]

        for j in range(_NCH):
            tap = tap_ref[:, j * chunk:(j + 1) * chunk].astype(jnp.bfloat16)
            acc = jax.lax.dot_general(
                tap, w_ref[...], (((0,), (0,)), ((), ())),
                preferred_element_type=jnp.float32)

            @pl.when(b >= _NPAR)
            def _wait_slot(j=j):
                pltpu.make_async_copy(
                    obuf.at[par, j], obuf.at[par, j], sems.at[par, j]).wait()
            obuf[par, j] = acc
            pltpu.make_async_copy(
                obuf.at[par, j], o_hbm.at[b, pl.ds(j * chunk, chunk), :],
                sems.at[par, j]).start()

        @pl.when(b == nb - 1)
        def _drain():
            for p in range(_NPAR):
                for j in range(_NCH):
                    pltpu.make_async_copy(
                        obuf.at[p, j], obuf.at[p, j], sems.at[p, j]).wait()
    return body


def kernel(x, conv_w, conv_b, leftout_w, leftout_b):
    b_sz, length, c_in = x.shape
    n_k, mp1, ksize = conv_w.shape
    n_left = leftout_w.shape[0]
    d_model = n_k * c_in + n_left
    n_groups = mp1 * ksize
    kdim = n_groups * 8

    w18 = conv_w[:, ::-1, :].reshape(n_k, n_groups)
    l18 = leftout_w[:, ::-1, :].reshape(n_left, n_groups)
    eye = jnp.eye(8, c_in, dtype=jnp.float32)
    blk = eye[None, :, :, None] * jnp.transpose(w18)[:, None, None, :]
    blk = blk.reshape(n_groups, 8, c_in * n_k)
    last = (jnp.arange(8) == (c_in - 1)).astype(jnp.float32)[None, :, None]
    last = last * jnp.transpose(l18)[:, None, :]
    w_mat = jnp.concatenate([blk, last], axis=-1).reshape(kdim, d_model)
    bias = jnp.concatenate([jnp.tile(conv_b, c_in), leftout_b])
    w_mat = w_mat.at[7, :].set(bias)
    w_mat = w_mat.astype(jnp.bfloat16)

    g_idx = jnp.arange(n_groups)
    s_of_g = (g_idx - (n_groups - 2) + 1) % 3
    t_head = jnp.arange(128)
    head = jnp.where(
        s_of_g[:, None] == 0, ((t_head >= 16) | (t_head == 0))[None, :],
        jnp.where(s_of_g[:, None] == 1, (t_head >= 15)[None, :],
                  (t_head >= 14)[None, :])).astype(jnp.float32)
    hm = jnp.repeat(head, 8, axis=0)
    tail = jnp.where((s_of_g[:, None] == 2) & (t_head == 127)[None, :],
                     0.0, 1.0)
    tm = jnp.repeat(tail, 8, axis=0)
    hm = hm.at[7, :].set(1.0)
    tm = tm.at[7, :].set(1.0)

    xt = jnp.transpose(x, (0, 2, 1))
    xt = jnp.pad(xt, ((0, 0), (0, 8 - c_in), (0, 0)))

    chunk = length // _NCH
    out = pl.pallas_call(
        _build_tap_kernel(n_groups, c_in, length, d_model),
        grid=(b_sz,),
        in_specs=[
            pl.BlockSpec((1, 8, length), lambda b: (b, 0, 0)),
            pl.BlockSpec((kdim, d_model), lambda b: (0, 0)),
            pl.BlockSpec((kdim, 128), lambda b: (0, 0)),
            pl.BlockSpec((kdim, 128), lambda b: (0, 0)),
        ],
        out_specs=pl.BlockSpec(memory_space=pl.ANY),
        out_shape=jax.ShapeDtypeStruct((b_sz, length, d_model), jnp.float32),
        scratch_shapes=[
            pltpu.VMEM((kdim, length), jnp.float32),
            pltpu.VMEM((_NPAR, _NCH, chunk, d_model), jnp.float32),
            pltpu.SemaphoreType.DMA((_NPAR, _NCH)),
        ],
        compiler_params=pltpu.CompilerParams(
            dimension_semantics=("arbitrary",),
            vmem_limit_bytes=56 * 1024 * 1024,
        ),
    )(xt, w_mat, hm, tm)
    return out


# speedup vs baseline: 1.1894x; 1.1894x over previous
import jax
import jax.numpy as jnp
from jax.experimental import pallas as pl
from jax.experimental.pallas import tpu as pltpu

_BPB = 2


def _build_tap_kernel(n_groups, c_in, length):
    def body(xt_ref, w_ref, hm_ref, tm_ref, o_ref, tap_ref):
        sub = jax.lax.broadcasted_iota(jnp.int32, (8, length), 0)
        for bb in range(_BPB):
            xt = xt_ref[bb]
            for g in range(n_groups):
                d = g - (n_groups - 2)
                rolled = pltpu.roll(xt, (-d) % length, axis=1)
                if g == 0:
                    rolled = jnp.where(sub == 7, 1.0, rolled)
                tap_ref[g * 8:(g + 1) * 8, :] = rolled
            tap_ref[:, :128] = tap_ref[:, :128] * hm_ref[...]
            tap_ref[:, length - 128:] = tap_ref[:, length - 128:] * tm_ref[...]
            tap = tap_ref[...].astype(jnp.bfloat16)
            o_ref[bb] = jax.lax.dot_general(
                tap, w_ref[...], (((0,), (0,)), ((), ())),
                preferred_element_type=jnp.float32)
    return body


def kernel(x, conv_w, conv_b, leftout_w, leftout_b):
    b_sz, length, c_in = x.shape
    n_k, mp1, ksize = conv_w.shape
    n_left = leftout_w.shape[0]
    d_model = n_k * c_in + n_left
    n_groups = mp1 * ksize
    kdim = n_groups * 8

    w18 = conv_w[:, ::-1, :].reshape(n_k, n_groups)
    l18 = leftout_w[:, ::-1, :].reshape(n_left, n_groups)
    eye = jnp.eye(8, c_in, dtype=jnp.float32)
    blk = eye[None, :, :, None] * jnp.transpose(w18)[:, None, None, :]
    blk = blk.reshape(n_groups, 8, c_in * n_k)
    last = (jnp.arange(8) == (c_in - 1)).astype(jnp.float32)[None, :, None]
    last = last * jnp.transpose(l18)[:, None, :]
    w_mat = jnp.concatenate([blk, last], axis=-1).reshape(kdim, d_model)
    bias = jnp.concatenate([jnp.tile(conv_b, c_in), leftout_b])
    w_mat = w_mat.at[7, :].set(bias)
    w_mat = w_mat.astype(jnp.bfloat16)

    g_idx = jnp.arange(n_groups)
    s_of_g = (g_idx - (n_groups - 2) + 1) % 3
    t_head = jnp.arange(128)
    head = jnp.where(
        s_of_g[:, None] == 0, ((t_head >= 16) | (t_head == 0))[None, :],
        jnp.where(s_of_g[:, None] == 1, (t_head >= 15)[None, :],
                  (t_head >= 14)[None, :])).astype(jnp.float32)
    hm = jnp.repeat(head, 8, axis=0)
    tail = jnp.where((s_of_g[:, None] == 2) & (t_head == 127)[None, :],
                     0.0, 1.0)
    tm = jnp.repeat(tail, 8, axis=0)
    hm = hm.at[7, :].set(1.0)
    tm = tm.at[7, :].set(1.0)

    xt = jnp.transpose(x, (0, 2, 1))
    xt = jnp.pad(xt, ((0, 0), (0, 8 - c_in), (0, 0)))

    out = pl.pallas_call(
        _build_tap_kernel(n_groups, c_in, length),
        grid=(b_sz // _BPB,),
        in_specs=[
            pl.BlockSpec((_BPB, 8, length), lambda b: (b, 0, 0)),
            pl.BlockSpec((kdim, d_model), lambda b: (0, 0)),
            pl.BlockSpec((kdim, 128), lambda b: (0, 0)),
            pl.BlockSpec((kdim, 128), lambda b: (0, 0)),
        ],
        out_specs=pl.BlockSpec((_BPB, length, d_model), lambda b: (b, 0, 0)),
        out_shape=jax.ShapeDtypeStruct((b_sz, length, d_model), jnp.float32),
        scratch_shapes=[pltpu.VMEM((kdim, length), jnp.float32)],
        compiler_params=pltpu.CompilerParams(
            dimension_semantics=("arbitrary",),
            vmem_limit_bytes=56 * 1024 * 1024,
        ),
    )(xt, w_mat, hm, tm)
    return out
